# baseline (speedup 1.0000x reference)
import functools

import jax
import jax.numpy as jnp
from jax import lax
from jax.experimental import pallas as pl
from jax.experimental.pallas import tpu as pltpu
from jax.experimental.pallas import tpu_sc as plsc

_NC = 2
_NS = 16
_NW = _NC * _NS
_L = 16
_LANES = 128
_ROWS = 4096 * 100
_RPW = _ROWS // _NW
_CHUNK = 256
_NCHUNK = _RPW // _CHUNK
_CW = _CHUNK * _LANES


def _sc_permute(xf, perm):
    mesh = plsc.VectorSubcoreMesh(core_axis_name="c", subcore_axis_name="s")

    @functools.partial(
        pl.kernel,
        out_type=jax.ShapeDtypeStruct((_ROWS * _LANES,), jnp.float32),
        mesh=mesh,
        scratch_types=[
            pltpu.VMEM((_LANES,), jnp.int32),
            pltpu.VMEM((_CW,), jnp.float32),
            pltpu.VMEM((_CW,), jnp.float32),
        ],
        compiler_params=pltpu.CompilerParams(needs_layout_passes=False),
    )
    def body(x_hbm, perm_hbm, y_hbm, perm_v, in_v, out_v):
        wid = lax.axis_index("s") * _NC + lax.axis_index("c")
        base = wid * _RPW * _LANES
        pltpu.sync_copy(perm_hbm, perm_v)
        lane_idx = [perm_v[pl.ds(_L * k, _L)] for k in range(_LANES // _L)]

        def chunk_body(ci, carry):
            off = base + ci * _CW
            pltpu.sync_copy(x_hbm.at[pl.ds(off, _CW)], in_v)

            def row_body(r, rc):
                rbase = jnp.full((_L,), r * _LANES, jnp.int32)
                for k in range(_LANES // _L):
                    vals = plsc.load_gather(in_v, [rbase + lane_idx[k]])
                    out_v[pl.ds(r * _LANES + _L * k, _L)] = vals
                return rc

            lax.fori_loop(0, _CHUNK, row_body, 0)

            pltpu.sync_copy(out_v, y_hbm.at[pl.ds(off, _CW)])
            return carry

        lax.fori_loop(0, _NCHUNK, chunk_body, 0)

    return body(xf, perm)


def kernel(x, permutation):
    xf = x.reshape(_ROWS * _LANES)
    y = _sc_permute(xf, permutation)
    return y.reshape(x.shape), jnp.zeros(x.shape[:-1], x.dtype)


# speedup vs baseline: 1.1574x; 1.1574x over previous
import functools

import jax
import jax.numpy as jnp
from jax import lax
from jax.experimental import pallas as pl
from jax.experimental.pallas import tpu as pltpu
from jax.experimental.pallas import tpu_sc as plsc

_NC = 2
_NS = 16
_NW = _NC * _NS
_L = 16
_LANES = 128
_ROWS = 4096 * 100
_RPW = _ROWS // _NW
_CHUNK = 128
_NCHUNK = _RPW // _CHUNK
_CW = _CHUNK * _LANES


def _sc_permute(xf, perm):
    mesh = plsc.VectorSubcoreMesh(core_axis_name="c", subcore_axis_name="s")

    @functools.partial(
        pl.kernel,
        out_type=jax.ShapeDtypeStruct((_ROWS * _LANES,), jnp.float32),
        mesh=mesh,
        scratch_types=[
            pltpu.VMEM((_LANES,), jnp.int32),
            pltpu.VMEM((_CW,), jnp.float32),
            pltpu.VMEM((_CW,), jnp.float32),
            pltpu.VMEM((_CW,), jnp.float32),
            pltpu.VMEM((_CW,), jnp.float32),
            pltpu.SemaphoreType.DMA,
            pltpu.SemaphoreType.DMA,
            pltpu.SemaphoreType.DMA,
            pltpu.SemaphoreType.DMA,
        ],
        compiler_params=pltpu.CompilerParams(needs_layout_passes=False),
    )
    def body(x_hbm, perm_hbm, y_hbm, perm_v,
             in0, in1, out0, out1, si0, si1, so0, so1):
        wid = lax.axis_index("s") * _NC + lax.axis_index("c")
        base = wid * _RPW * _LANES
        pltpu.sync_copy(perm_hbm, perm_v)
        lane_idx = [perm_v[pl.ds(_L * k, _L)] for k in range(_LANES // _L)]

        ins, outs = (in0, in1), (out0, out1)
        sis, sos = (si0, si1), (so0, so1)

        def start_in(ci, b):
            pltpu.async_copy(x_hbm.at[pl.ds(base + ci * _CW, _CW)],
                             ins[b], sis[b])

        def wait_in(b):
            pltpu.make_async_copy(x_hbm.at[pl.ds(0, _CW)],
                                  ins[b], sis[b]).wait()

        def start_out(ci, b):
            pltpu.async_copy(outs[b],
                             y_hbm.at[pl.ds(base + ci * _CW, _CW)], sos[b])

        def wait_out(b):
            pltpu.make_async_copy(outs[b],
                                  y_hbm.at[pl.ds(0, _CW)], sos[b]).wait()

        def permute_chunk(in_v, out_v):
            def row_body(r, rc):
                rbase = jnp.full((_L,), r * _LANES, jnp.int32)
                for k in range(_LANES // _L):
                    vals = plsc.load_gather(in_v, [rbase + lane_idx[k]])
                    out_v[pl.ds(r * _LANES + _L * k, _L)] = vals
                return rc

            lax.fori_loop(0, _CHUNK, row_body, 0, unroll=4)

        start_in(0, 0)
        start_in(1, 1)

        def chunk_pair(i, carry):
            for b in (0, 1):
                ci = 2 * i + b
                wait_in(b)

                @pl.when(i > 0)
                def _():
                    wait_out(b)

                permute_chunk(ins[b], outs[b])
                start_out(ci, b)

                @pl.when(ci + 2 < _NCHUNK)
                def _():
                    start_in(ci + 2, b)

            return carry

        lax.fori_loop(0, _NCHUNK // 2, chunk_pair, 0)
        wait_out(0)
        wait_out(1)

    return body(xf, perm)


def kernel(x, permutation):
    xf = x.reshape(_ROWS * _LANES)
    y = _sc_permute(xf, permutation)
    return y.reshape(x.shape), jnp.zeros(x.shape[:-1], x.dtype)
